# baseline (speedup 1.0000x reference)
import jax
import jax.numpy as jnp
from jax.experimental import pallas as pl
from jax.experimental.pallas import tpu as pltpu

M = 10000
K = 10000
N = 128
BS = 200
BMD = 400
NBIG = 24
NSTEPS = M // BS
SPLIT = 2 * NBIG


def _body(adj_hbm, x_ref, out_ref, buf, sems):
    i = pl.program_id(0)

    def big_copy(b, slot):
        return pltpu.make_async_copy(
            adj_hbm.at[pl.ds(b * BMD, BMD), :], buf.at[slot], sems.at[slot]
        )

    def small_copy(s, slot):
        return pltpu.make_async_copy(
            adj_hbm.at[pl.ds(NBIG * BMD + s * BS, BS), :],
            buf.at[slot, pl.ds(0, BS)],
            sems.at[slot],
        )

    b = i // 2
    even = i % 2 == 0

    @pl.when(i == 0)
    def _prologue():
        big_copy(0, 0).start()
        big_copy(1, 1).start()

    @pl.when(even & (b >= 1) & (b + 1 < NBIG))
    def _prefetch_big():
        big_copy(b + 1, (b + 1) % 2).start()

    @pl.when(i == 2 * (NBIG - 1))
    def _prefetch_small0():
        small_copy(0, 0).start()

    @pl.when(i == SPLIT)
    def _prefetch_small1():
        small_copy(1, 1).start()

    @pl.when(even & (i < SPLIT))
    def _wait_big():
        big_copy(b, b % 2).wait()

    @pl.when(i >= SPLIT)
    def _wait_small():
        small_copy(i - SPLIT, i - SPLIT).wait()

    slot = jnp.where(i < SPLIT, b % 2, i - SPLIT)
    off = jnp.where(i < SPLIT, (i % 2) * BS, 0)
    off = pl.multiple_of(off, 8)
    a = buf[slot, pl.ds(off, BS), :]
    out_ref[...] = jax.lax.dot_general(
        a,
        x_ref[...],
        (((1,), (0,)), ((), ())),
        precision=jax.lax.Precision.DEFAULT,
        preferred_element_type=jnp.float32,
    )


def kernel(x, adj):
    return pl.pallas_call(
        _body,
        grid=(NSTEPS,),
        in_specs=[
            pl.BlockSpec(memory_space=pl.ANY),
            pl.BlockSpec((K, N), lambda i: (0, 0)),
        ],
        out_specs=pl.BlockSpec((BS, N), lambda i: (i, 0)),
        out_shape=jax.ShapeDtypeStruct((M, N), jnp.float32),
        scratch_shapes=[
            pltpu.VMEM((2, BMD, K), jnp.float32),
            pltpu.SemaphoreType.DMA((2,)),
        ],
        compiler_params=pltpu.CompilerParams(
            dimension_semantics=("arbitrary",),
        ),
    )(adj, x)


# speedup vs baseline: 1.0054x; 1.0054x over previous
import jax
import jax.numpy as jnp
from jax.experimental import pallas as pl
from jax.experimental.pallas import tpu as pltpu

M = 10000
K = 10000
N = 128
BM = 400


def _matmul_block(adj_ref, x_ref, out_ref):
    out_ref[...] = jax.lax.dot_general(
        adj_ref[...],
        x_ref[...],
        (((1,), (0,)), ((), ())),
        precision=jax.lax.Precision.DEFAULT,
        preferred_element_type=jnp.float32,
    )


def kernel(x, adj):
    return pl.pallas_call(
        _matmul_block,
        grid=(M // BM,),
        in_specs=[
            pl.BlockSpec((BM, K), lambda i: (i, 0)),
            pl.BlockSpec((K, N), lambda i: (0, 0)),
        ],
        out_specs=pl.BlockSpec((BM, N), lambda i: (i, 0)),
        out_shape=jax.ShapeDtypeStruct((M, N), jnp.float32),
        compiler_params=pltpu.CompilerParams(
            dimension_semantics=("parallel",),
        ),
    )(adj, x)
